# baseline (speedup 1.0000x reference)
import functools

import jax
import jax.numpy as jnp
from jax import lax
from jax.experimental import pallas as pl
from jax.experimental.pallas import tpu as pltpu
from jax.experimental.pallas import tpu_sc as plsc

MAXLEN_ = 200
EMBED_ = 64
BATCH_ = 1024
NWORK_ = 32
BBLK_ = 128
NBBLK_ = BATCH_ // BBLK_
NTASK_ = MAXLEN_ * NBBLK_
TPW_ = NTASK_ // NWORK_
SBLK_ = 16
XPAD_ = 208


def _emb_kernel(x_hbm, tok_hbm, pos_hbm, out_hbm, pos_v, xcols,
                idx0, idx1, tok0, tok1, tr0, tr1, gs0, gs1, os0, os1):
    nc = 2
    wid = lax.axis_index("s") * nc + lax.axis_index("c")
    t0 = wid * TPW_
    s0 = (t0 // NBBLK_) // 8 * 8

    pltpu.sync_copy(pos_hbm, pos_v)
    pltpu.sync_copy(x_hbm.at[:, pl.ds(s0, SBLK_)],
                    xcols.at[:, pl.ds(0, SBLK_)])

    idxs = (idx0, idx1)
    toks = (tok0, tok1)
    trs = (tr0, tr1)
    gsems = (gs0, gs1)
    osems = (os0, os1)

    lanes = lax.iota(jnp.int32, 16)
    ti_c = [(q * 16 + lanes) >> 3 for q in range(EMBED_ // 16)]
    r_c = [(q * 16 + lanes) & 7 for q in range(EMBED_ // 16)]

    def start_gather(t, b):
        s = t // NBBLK_
        tj = lax.rem(t, NBBLK_)
        cvec = jnp.broadcast_to(s - s0, (16,))
        base = jnp.broadcast_to(tj * BBLK_, (16,)) + lanes
        for q in range(BBLK_ // 16):
            vals = plsc.load_gather(xcols, [base + q * 16, cvec])
            idxs[b][pl.ds(q * 16, 16)] = vals
        pltpu.async_copy(tok_hbm.at[idxs[b]], toks[b], gsems[b])

    def wait_gather(b):
        pltpu.make_async_copy(tok_hbm.at[idxs[b]], toks[b], gsems[b]).wait()

    def start_out(t, b):
        s = t // NBBLK_
        tj = lax.rem(t, NBBLK_)
        pltpu.async_copy(trs[b].at[:, :, pl.ds(0, BBLK_)],
                         out_hbm.at[s, slice(None), tj], osems[b])

    def wait_out(b):
        pltpu.make_async_copy(trs[b].at[:, :, pl.ds(0, BBLK_)],
                              out_hbm.at[0, slice(None), 0], osems[b]).wait()

    start_gather(t0, 0)

    def super_body(kk, carry):
        for b in (0, 1):
            i = kk * 2 + b
            t = t0 + i
            nb = 1 - b
            wait_gather(b)

            @pl.when(i + 1 < TPW_)
            def _():
                start_gather(t + 1, nb)

            @pl.when(i >= 2)
            def _():
                wait_out(b)

            s = t // NBBLK_
            tok = toks[b]
            tr = trs[b]
            prow = [pos_v[s, pl.ds(q * 16, 16)] for q in range(EMBED_ // 16)]

            @plsc.parallel_loop(0, BBLK_, unroll=4,
                                carry=jnp.broadcast_to(0, (16,)))
            def j_loop(j, jvec):
                for q in range(EMBED_ // 16):
                    vals = tok[j, pl.ds(q * 16, 16)] + prow[q]
                    plsc.store_scatter(tr, [ti_c[q], r_c[q], jvec], vals)
                return jvec + 1

            start_out(t, b)
        return carry

    lax.fori_loop(0, TPW_ // 2, super_body, 0)
    wait_out(0)
    wait_out(1)


def kernel(x, token_table, pos_table):
    batch, seqlen = x.shape
    xp = jnp.pad(x.astype(jnp.int32), ((0, 0), (0, XPAD_ - seqlen)))

    mesh = plsc.VectorSubcoreMesh(core_axis_name="c", subcore_axis_name="s")
    run = functools.partial(
        pl.kernel,
        mesh=mesh,
        compiler_params=pltpu.CompilerParams(
            use_tc_tiling_on_sc=False, needs_layout_passes=False),
        out_type=jax.ShapeDtypeStruct(
            (MAXLEN_, EMBED_ // 8, NBBLK_, 8, BBLK_), jnp.float32),
        scratch_types=[
            pltpu.VMEM((MAXLEN_, EMBED_), jnp.float32),
            pltpu.VMEM((BATCH_, SBLK_ + 1), jnp.int32),
            pltpu.VMEM((BBLK_,), jnp.int32),
            pltpu.VMEM((BBLK_,), jnp.int32),
            pltpu.VMEM((BBLK_, EMBED_), jnp.float32),
            pltpu.VMEM((BBLK_, EMBED_), jnp.float32),
            pltpu.VMEM((EMBED_ // 8, 8, BBLK_ + 1), jnp.float32),
            pltpu.VMEM((EMBED_ // 8, 8, BBLK_ + 1), jnp.float32),
            pltpu.SemaphoreType.DMA,
            pltpu.SemaphoreType.DMA,
            pltpu.SemaphoreType.DMA,
            pltpu.SemaphoreType.DMA,
        ],
    )(_emb_kernel)
    out5 = run(xp, token_table, pos_table)
    out = jnp.transpose(out5, (0, 1, 3, 2, 4)).reshape(MAXLEN_, EMBED_, BATCH_)
    return jnp.transpose(out, (2, 0, 1))


# speedup vs baseline: 1.0010x; 1.0010x over previous
import functools

import jax
import jax.numpy as jnp
from jax import lax
from jax.experimental import pallas as pl
from jax.experimental.pallas import tpu as pltpu
from jax.experimental.pallas import tpu_sc as plsc

MAXLEN_ = 200
EMBED_ = 64
BATCH_ = 1024
NWORK_ = 32
BBLK_ = 128
NBBLK_ = BATCH_ // BBLK_
NTASK_ = MAXLEN_ * NBBLK_
TPW_ = NTASK_ // NWORK_
SBLK_ = 16
XPAD_ = 208


def _emb_kernel(x_hbm, tok_hbm, pos_hbm, out_hbm, pos_v, xcols,
                idx0, idx1, tok0, tok1, tr0, tr1, gs0, gs1, os0, os1):
    nc = 2
    wid = lax.axis_index("s") * nc + lax.axis_index("c")
    t0 = wid * TPW_
    s0 = jnp.minimum((t0 // NBBLK_) // 8 * 8, MAXLEN_ - SBLK_)

    pltpu.sync_copy(pos_hbm, pos_v)
    pltpu.sync_copy(x_hbm.at[:, pl.ds(s0, SBLK_)],
                    xcols.at[:, pl.ds(0, SBLK_)])

    idxs = (idx0, idx1)
    toks = (tok0, tok1)
    trs = (tr0, tr1)
    gsems = (gs0, gs1)
    osems = (os0, os1)

    lanes = lax.iota(jnp.int32, 16)
    ti_c = [(q * 16 + lanes) >> 3 for q in range(EMBED_ // 16)]
    r_c = [(q * 16 + lanes) & 7 for q in range(EMBED_ // 16)]

    def start_gather(t, b):
        s = t // NBBLK_
        tj = lax.rem(t, NBBLK_)
        cvec = jnp.broadcast_to(s - s0, (16,))
        base = jnp.broadcast_to(tj * BBLK_, (16,)) + lanes
        for q in range(BBLK_ // 16):
            vals = plsc.load_gather(xcols, [base + q * 16, cvec])
            idxs[b][pl.ds(q * 16, 16)] = vals
        pltpu.async_copy(tok_hbm.at[idxs[b]], toks[b], gsems[b])

    def wait_gather(b):
        pltpu.make_async_copy(tok_hbm.at[idxs[b]], toks[b], gsems[b]).wait()

    def start_out(t, b):
        s = t // NBBLK_
        tj = lax.rem(t, NBBLK_)
        pltpu.async_copy(trs[b].at[:, :, pl.ds(0, BBLK_)],
                         out_hbm.at[s, slice(None), tj], osems[b])

    def wait_out(b):
        pltpu.make_async_copy(trs[b].at[:, :, pl.ds(0, BBLK_)],
                              out_hbm.at[0, slice(None), 0], osems[b]).wait()

    start_gather(t0, 0)

    def super_body(kk, carry):
        for b in (0, 1):
            i = kk * 2 + b
            t = t0 + i
            nb = 1 - b
            wait_gather(b)

            @pl.when(i + 1 < TPW_)
            def _():
                start_gather(t + 1, nb)

            @pl.when(i >= 2)
            def _():
                wait_out(b)

            s = t // NBBLK_
            tok = toks[b]
            tr = trs[b]
            prow = [pos_v[s, pl.ds(q * 16, 16)] for q in range(EMBED_ // 16)]

            @plsc.parallel_loop(0, BBLK_, unroll=4,
                                carry=jnp.broadcast_to(0, (16,)))
            def j_loop(j, jvec):
                for q in range(EMBED_ // 16):
                    vals = tok[j, pl.ds(q * 16, 16)] + prow[q]
                    plsc.store_scatter(tr, [ti_c[q], r_c[q], jvec], vals)
                return jvec + 1

            start_out(t, b)
        return carry

    lax.fori_loop(0, TPW_ // 2, super_body, 0)
    wait_out(0)
    wait_out(1)


def kernel(x, token_table, pos_table):
    batch, seqlen = x.shape
    xp = x.astype(jnp.int32)

    mesh = plsc.VectorSubcoreMesh(core_axis_name="c", subcore_axis_name="s")
    run = functools.partial(
        pl.kernel,
        mesh=mesh,
        compiler_params=pltpu.CompilerParams(
            use_tc_tiling_on_sc=False, needs_layout_passes=False),
        out_type=jax.ShapeDtypeStruct(
            (MAXLEN_, EMBED_ // 8, NBBLK_, 8, BBLK_), jnp.float32),
        scratch_types=[
            pltpu.VMEM((MAXLEN_, EMBED_), jnp.float32),
            pltpu.VMEM((BATCH_, SBLK_ + 1), jnp.int32),
            pltpu.VMEM((BBLK_,), jnp.int32),
            pltpu.VMEM((BBLK_,), jnp.int32),
            pltpu.VMEM((BBLK_, EMBED_), jnp.float32),
            pltpu.VMEM((BBLK_, EMBED_), jnp.float32),
            pltpu.VMEM((EMBED_ // 8, 8, BBLK_ + 1), jnp.float32),
            pltpu.VMEM((EMBED_ // 8, 8, BBLK_ + 1), jnp.float32),
            pltpu.SemaphoreType.DMA,
            pltpu.SemaphoreType.DMA,
            pltpu.SemaphoreType.DMA,
            pltpu.SemaphoreType.DMA,
        ],
    )(_emb_kernel)
    out5 = run(xp, token_table, pos_table)
    out = jnp.transpose(out5, (0, 1, 3, 2, 4)).reshape(MAXLEN_, EMBED_, BATCH_)
    return jnp.transpose(out, (2, 0, 1))


# speedup vs baseline: 1.0456x; 1.0446x over previous
import functools

import jax
import jax.numpy as jnp
from jax import lax
from jax.experimental import pallas as pl
from jax.experimental.pallas import tpu as pltpu
from jax.experimental.pallas import tpu_sc as plsc

MAXLEN_ = 200
EMBED_ = 64
BATCH_ = 1024
NWORK_ = 32
BBLK_ = 128
NBBLK_ = BATCH_ // BBLK_
NTASK_ = MAXLEN_ * NBBLK_
TPW_ = NTASK_ // NWORK_
SBLK_ = 16
XPAD_ = 208


def _emb_kernel(xq_hbm, tok_hbm, pos_hbm, out_hbm, pos_v, idx_v,
                tok0, tok1, tr0, tr1, gs0, gs1, os0, os1):
    nc = 2
    wid = lax.axis_index("s") * nc + lax.axis_index("c")
    p0 = wid * TPW_

    pltpu.sync_copy(pos_hbm, pos_v)
    pltpu.sync_copy(xq_hbm.at[pl.ds(p0, TPW_)], idx_v)

    toks = (tok0, tok1)
    trs = (tr0, tr1)
    gsems = (gs0, gs1)
    osems = (os0, os1)

    lanes = lax.iota(jnp.int32, 16)
    ti_c = [(q * 16 + lanes) >> 3 for q in range(EMBED_ // 16)]
    r_c = [(q * 16 + lanes) & 7 for q in range(EMBED_ // 16)]

    def task_pos(p):
        s = ((p >> 6) << 3) | (p & 7)
        tj = (p >> 3) & 7
        return s, tj

    def start_gather(i, b):
        pltpu.async_copy(tok_hbm.at[idx_v.at[i]], toks[b], gsems[b])

    def wait_gather(b):
        pltpu.make_async_copy(tok_hbm.at[idx_v.at[0]], toks[b], gsems[b]).wait()

    def start_out(p, b):
        s, tj = task_pos(p)
        pltpu.async_copy(trs[b].at[:, :, pl.ds(0, BBLK_)],
                         out_hbm.at[s, slice(None), tj], osems[b])

    def wait_out(b):
        pltpu.make_async_copy(trs[b].at[:, :, pl.ds(0, BBLK_)],
                              out_hbm.at[0, slice(None), 0], osems[b]).wait()

    start_gather(0, 0)

    def super_body(kk, carry):
        for b in (0, 1):
            i = kk * 2 + b
            p = p0 + i
            nb = 1 - b
            wait_gather(b)

            @pl.when(i + 1 < TPW_)
            def _():
                start_gather(i + 1, nb)

            @pl.when(i >= 2)
            def _():
                wait_out(b)

            s, _tj = task_pos(p)
            tok = toks[b]
            tr = trs[b]
            prow = [pos_v[s, pl.ds(q * 16, 16)] for q in range(EMBED_ // 16)]

            @plsc.parallel_loop(0, BBLK_, unroll=4,
                                carry=jnp.broadcast_to(0, (16,)))
            def j_loop(j, jvec):
                for q in range(EMBED_ // 16):
                    vals = tok[j, pl.ds(q * 16, 16)] + prow[q]
                    plsc.store_scatter(tr, [ti_c[q], r_c[q], jvec], vals)
                return jvec + 1

            start_out(p, b)
        return carry

    lax.fori_loop(0, TPW_ // 2, super_body, 0)
    wait_out(0)
    wait_out(1)


def kernel(x, token_table, pos_table):
    batch, seqlen = x.shape
    xq = jnp.transpose(x.astype(jnp.int32), (1, 0))
    xq = jnp.transpose(xq.reshape(MAXLEN_ // 8, 8, NBBLK_, BBLK_), (0, 2, 1, 3))
    xq = xq.reshape(NTASK_, BBLK_)

    mesh = plsc.VectorSubcoreMesh(core_axis_name="c", subcore_axis_name="s")
    run = functools.partial(
        pl.kernel,
        mesh=mesh,
        compiler_params=pltpu.CompilerParams(
            use_tc_tiling_on_sc=False, needs_layout_passes=False),
        out_type=jax.ShapeDtypeStruct(
            (MAXLEN_, EMBED_ // 8, NBBLK_, 8, BBLK_), jnp.float32),
        scratch_types=[
            pltpu.VMEM((MAXLEN_, EMBED_), jnp.float32),
            pltpu.VMEM((TPW_, BBLK_), jnp.int32),
            pltpu.VMEM((BBLK_, EMBED_), jnp.float32),
            pltpu.VMEM((BBLK_, EMBED_), jnp.float32),
            pltpu.VMEM((EMBED_ // 8, 8, BBLK_ + 1), jnp.float32),
            pltpu.VMEM((EMBED_ // 8, 8, BBLK_ + 1), jnp.float32),
            pltpu.SemaphoreType.DMA,
            pltpu.SemaphoreType.DMA,
            pltpu.SemaphoreType.DMA,
            pltpu.SemaphoreType.DMA,
        ],
    )(_emb_kernel)
    out5 = run(xq, token_table, pos_table)
    out = jnp.transpose(out5, (0, 1, 3, 2, 4)).reshape(MAXLEN_, EMBED_, BATCH_)
    return jnp.transpose(out, (2, 0, 1))
